# Initial kernel scaffold; baseline (speedup 1.0000x reference)
import jax
import jax.numpy as jnp
from jax.experimental import pallas as pl


def kernel(x, node_labels, Param_W, Param_b):
    raise NotImplementedError("write your pallas kernel here")


# speedup vs baseline: 4.3690x; 4.3690x over previous
import functools

import jax
import jax.numpy as jnp
from jax import lax
from jax.experimental import pallas as pl
from jax.experimental.pallas import tpu as pltpu
from jax.experimental.pallas import tpu_sc as plsc

_C = 1
_O = 64
_L = 50
_N = 100000
_F = 128

_CHUNK = 128
_NFULL = _N // _CHUNK
_TAIL = _N - _NFULL * _CHUNK


def _seg_sum_sc(x, labels):
    info = plsc.get_sparse_core_info()
    nc, ns = info.num_cores, info.num_subcores
    nw = nc * ns

    mesh = plsc.VectorSubcoreMesh(core_axis_name="c", subcore_axis_name="s")

    @functools.partial(
        pl.kernel,
        out_type=jax.ShapeDtypeStruct((nc, _L, _F), jnp.float32),
        mesh=mesh,
        scratch_types=[
            pltpu.VMEM((_CHUNK, _F), jnp.float32),
            pltpu.VMEM((_CHUNK,), jnp.int32),
            pltpu.VMEM((_TAIL, _F), jnp.float32),
            pltpu.VMEM((_TAIL,), jnp.int32),
            pltpu.VMEM((_L, _F), jnp.float32),
            pltpu.VMEM_SHARED((_L, _F), jnp.float32),
        ],
    )
    def seg_kernel(x_hbm, lbl_hbm, out_hbm, x_v, lbl_v, xt_v, lblt_v, zero_v, acc_sh):
        cid = lax.axis_index("c")
        sid = lax.axis_index("s")
        wid = sid * nc + cid

        @pl.when(sid == 0)
        def _():
            @pl.loop(0, _L)
            def _(l):
                for j in range(_F // 16):
                    zero_v[l, pl.ds(j * 16, 16)] = jnp.zeros((16,), jnp.float32)
            pltpu.sync_copy(zero_v, acc_sh)

        plsc.subcore_barrier()

        nchunks = (_NFULL - 1 - wid) // nw + 1

        @pl.loop(0, nchunks)
        def _(i):
            c = wid + i * nw
            row0 = pl.multiple_of(c * _CHUNK, _CHUNK)
            pltpu.sync_copy(lbl_hbm.at[pl.ds(row0, _CHUNK)], lbl_v)
            pltpu.sync_copy(x_hbm.at[pl.ds(row0, _CHUNK), :], x_v)
            pltpu.sync_copy(x_v, acc_sh.at[lbl_v], add=True)

        @pl.when(wid == nw - 1)
        def _():
            t0 = _NFULL * _CHUNK
            pltpu.sync_copy(lbl_hbm.at[pl.ds(t0, _TAIL)], lblt_v)
            pltpu.sync_copy(x_hbm.at[pl.ds(t0, _TAIL), :], xt_v)
            pltpu.sync_copy(xt_v, acc_sh.at[lblt_v], add=True)

        plsc.subcore_barrier()

        @pl.when(sid == 0)
        def _():
            pltpu.sync_copy(acc_sh, out_hbm.at[cid])

    return seg_kernel(x, labels)


def _combine_tc(partials, w2, b):
    def tc_body(p_ref, w_ref, b_ref, o_ref):
        seg = p_ref[0] + p_ref[1]
        o_ref[...] = (
            jax.lax.dot(w_ref[...], seg, preferred_element_type=jnp.float32)
            + b_ref[0]
        )

    return pl.pallas_call(
        tc_body,
        out_shape=jax.ShapeDtypeStruct((_O, _F), jnp.float32),
    )(partials, w2, b)


def kernel(x, node_labels, Param_W, Param_b):
    labels = node_labels.astype(jnp.int32)
    partials = _seg_sum_sc(x, labels)
    w2 = Param_W.reshape(_O, _L)
    out = _combine_tc(partials, w2, Param_b)
    return out.reshape(_C, _O, _F)


# speedup vs baseline: 7.1280x; 1.6315x over previous
import functools

import jax
import jax.numpy as jnp
from jax import lax
from jax.experimental import pallas as pl
from jax.experimental.pallas import tpu as pltpu
from jax.experimental.pallas import tpu_sc as plsc

_C = 1
_O = 64
_L = 50
_N = 100000
_F = 128

_CHUNK = 128
_NFULL = _N // _CHUNK
_TAIL = _N - _NFULL * _CHUNK


def _seg_sum_sc(x, labels):
    info = plsc.get_sparse_core_info()
    nc, ns = info.num_cores, info.num_subcores
    nw = nc * ns

    nslots = (_NFULL + nw - 1) // nw
    last_cut = _NFULL - (nslots - 1) * nw

    mesh = plsc.VectorSubcoreMesh(core_axis_name="c", subcore_axis_name="s")

    @functools.partial(
        pl.kernel,
        out_type=jax.ShapeDtypeStruct((nc, _L, _F), jnp.float32),
        mesh=mesh,
        scratch_types=[
            pltpu.VMEM((2, _CHUNK, _F), jnp.float32),
            pltpu.VMEM((2, _CHUNK), jnp.int32),
            pltpu.VMEM((_TAIL, _F), jnp.float32),
            pltpu.VMEM((_TAIL,), jnp.int32),
            pltpu.VMEM((_L, _F), jnp.float32),
            pltpu.VMEM_SHARED((_L, _F), jnp.float32),
            pltpu.SemaphoreType.DMA,
            pltpu.SemaphoreType.DMA,
            pltpu.SemaphoreType.DMA,
            pltpu.SemaphoreType.DMA,
            pltpu.SemaphoreType.DMA,
        ],
    )
    def seg_kernel(x_hbm, lbl_hbm, out_hbm, x_v, lbl_v, xt_v, lblt_v, zero_v,
                   acc_sh, sx0, sx1, sl0, sl1, st):
        cid = lax.axis_index("c")
        sid = lax.axis_index("s")
        wid = sid * nc + cid
        sx = (sx0, sx1)
        sl = (sl0, sl1)

        def mk(i, b):
            c = wid + i * nw
            row0 = pl.multiple_of(c * _CHUNK, _CHUNK)
            dl = pltpu.make_async_copy(
                lbl_hbm.at[pl.ds(row0, _CHUNK)], lbl_v.at[b], sl[b])
            dx = pltpu.make_async_copy(
                x_hbm.at[pl.ds(row0, _CHUNK), :], x_v.at[b], sx[b])
            return dl, dx

        def start(i, b):
            dl, dx = mk(i, b)
            dl.start()
            dx.start()

        start(0, 0)
        t0 = _NFULL * _CHUNK
        dtl = pltpu.make_async_copy(lbl_hbm.at[pl.ds(t0, _TAIL)], lblt_v, st)
        dtx = pltpu.make_async_copy(x_hbm.at[pl.ds(t0, _TAIL), :], xt_v, st)

        @pl.when(wid == nw - 1)
        def _():
            dtl.start()
            dtx.start()

        @pl.when(sid == 0)
        def _():
            @pl.loop(0, _L)
            def _(l):
                for j in range(_F // 16):
                    zero_v[l, pl.ds(j * 16, 16)] = jnp.zeros((16,), jnp.float32)
            pltpu.sync_copy(zero_v, acc_sh)

        plsc.subcore_barrier()

        for i in range(nslots):
            b = i & 1

            def body(i=i, b=b):
                if i + 1 < nslots - 1:
                    start(i + 1, 1 - b)
                elif i + 1 == nslots - 1:
                    @pl.when(wid < last_cut)
                    def _():
                        start(i + 1, 1 - b)
                dl, dx = mk(i, b)
                dl.wait()
                dx.wait()
                pltpu.sync_copy(x_v.at[b], acc_sh.at[lbl_v.at[b]], add=True)

            if i < nslots - 1:
                body()
            else:
                pl.when(wid < last_cut)(body)

        @pl.when(wid == nw - 1)
        def _():
            dtl.wait()
            dtx.wait()
            pltpu.sync_copy(xt_v, acc_sh.at[lblt_v], add=True)

        plsc.subcore_barrier()

        @pl.when(sid == 0)
        def _():
            pltpu.sync_copy(acc_sh, out_hbm.at[cid])

    return seg_kernel(x, labels)


def _combine_tc(partials, w2, b):
    def tc_body(p_ref, w_ref, b_ref, o_ref):
        seg = p_ref[0] + p_ref[1]
        o_ref[...] = (
            jax.lax.dot(w_ref[...], seg, preferred_element_type=jnp.float32)
            + b_ref[0]
        )

    return pl.pallas_call(
        tc_body,
        out_shape=jax.ShapeDtypeStruct((_O, _F), jnp.float32),
    )(partials, w2, b)


def kernel(x, node_labels, Param_W, Param_b):
    labels = node_labels.astype(jnp.int32)
    partials = _seg_sum_sc(x, labels)
    w2 = Param_W.reshape(_O, _L)
    out = _combine_tc(partials, w2, Param_b)
    return out.reshape(_C, _O, _F)
